# baseline (speedup 1.0000x reference)
import functools

import jax
import jax.numpy as jnp
from jax import lax
from jax.experimental import pallas as pl
from jax.experimental.pallas import tpu as pltpu
from jax.experimental.pallas import tpu_sc as plsc

NC = 2
NS = 16
NW = NC * NS
LANES = 16
CHUNK = 128

_MESH = plsc.VectorSubcoreMesh(core_axis_name="c", subcore_axis_name="s")


def _fill_const(ref, rows, value):
    cols = ref.shape[1]

    @pl.loop(0, rows)
    def _(i):
        @pl.loop(0, cols, step=LANES)
        def _(j):
            ref[i, pl.ds(j, LANES)] = jnp.full((LANES,), value, jnp.float32)


ZROWS = 40


def _strided_row_chunks(total_rows, worker, n_workers, body):
    n_chunks = total_rows // ZROWS
    iters = (n_chunks + n_workers - 1) // n_workers

    @pl.loop(0, iters)
    def _(i):
        c = i * n_workers + worker

        @pl.when(c < n_chunks)
        def _():
            body(c * ZROWS)


def _make_degree_kernel(n_edges, n_nodes, feat):
    n_chunks = n_edges // CHUNK
    iters = (n_chunks + NS - 1) // NS
    assert n_nodes % ZROWS == 0

    @functools.partial(
        pl.kernel,
        out_type=jax.ShapeDtypeStruct((NC * n_nodes, feat), jnp.float32),
        mesh=_MESH,
        scratch_types=[
            pltpu.VMEM((CHUNK,), jnp.int32),
            pltpu.VMEM((CHUNK, feat), jnp.float32),
            pltpu.VMEM((ZROWS, feat), jnp.float32),
            pltpu.VMEM_SHARED((n_nodes, feat), jnp.float32),
        ],
    )
    def deg_kernel(idx_hbm, out_hbm, idx_v, ones_v, zeros_v, table):
        cid = lax.axis_index("c")
        sid = lax.axis_index("s")
        _fill_const(ones_v, CHUNK, 1.0)
        _fill_const(zeros_v, ZROWS, 0.0)

        _strided_row_chunks(
            n_nodes, sid, NS,
            lambda r: pltpu.sync_copy(zeros_v, table.at[pl.ds(r, ZROWS)]))
        plsc.subcore_barrier()

        @pl.loop(0, iters)
        def _(i):
            chunk = i * NS + sid

            @pl.when(chunk < n_chunks)
            def _():
                pltpu.sync_copy(
                    idx_hbm.at[pl.ds(cid * n_edges + chunk * CHUNK, CHUNK)],
                    idx_v)
                pltpu.sync_copy(ones_v, table.at[idx_v], add=True)

        plsc.subcore_barrier()
        _strided_row_chunks(
            n_nodes, sid, NS,
            lambda r: pltpu.sync_copy(
                table.at[pl.ds(r, ZROWS)],
                out_hbm.at[pl.ds(cid * n_nodes + r, ZROWS)]))

    return deg_kernel


def _make_aggregate_kernel(n_nodes, n_chunks, feat, n_acc):
    assert n_chunks % NW == 0 and n_nodes % ZROWS == 0
    cpw = n_chunks // NW
    n_phases = 2
    span = cpw // n_phases
    assert cpw % (2 * n_phases) == 0

    @functools.partial(
        pl.kernel,
        out_type=jax.ShapeDtypeStruct((NC * n_nodes, feat), jnp.float32),
        mesh=_MESH,
        scratch_types=[
            pltpu.VMEM((span, CHUNK), jnp.int32),
            pltpu.VMEM((span, CHUNK), jnp.int32),
            pltpu.VMEM((CHUNK, feat), jnp.float32),
            pltpu.VMEM((CHUNK, feat), jnp.float32),
            pltpu.VMEM((ZROWS, feat), jnp.float32),
            pltpu.VMEM_SHARED((n_acc, feat), jnp.float32),
            pltpu.SemaphoreType.DMA,
            pltpu.SemaphoreType.DMA,
            pltpu.SemaphoreType.DMA,
            pltpu.SemaphoreType.DMA,
        ],
    )
    def agg_kernel(x_hbm, src_hbm, dst_hbm, out_hbm,
                   sidx, didx, rows0, rows1, zeros_v, accum,
                   gsem0, gsem1, ssem0, ssem1):
        cid = lax.axis_index("c")
        sid = lax.axis_index("s")
        lw = cid * NS + sid
        _fill_const(zeros_v, ZROWS, 0.0)

        rows = (rows0, rows1)
        gsem = (gsem0, gsem1)
        ssem = (ssem0, ssem1)

        def load_idx_span(ph):
            base = lw * cpw + ph * span
            cp0 = pltpu.async_copy(src_hbm.at[pl.ds(base, span)], sidx, gsem0)
            cp1 = pltpu.async_copy(dst_hbm.at[pl.ds(base, span)], didx, gsem1)
            cp0.wait()
            cp1.wait()

        def start_gather(j, b):
            pltpu.make_async_copy(x_hbm.at[sidx.at[j]], rows[b], gsem[b]).start()

        def wait_gather(b):
            pltpu.make_async_copy(x_hbm.at[sidx.at[0]], rows[b], gsem[b]).wait()

        def start_scatter(j, b):
            pltpu.make_async_copy(rows[b], accum.at[didx.at[j]],
                                  ssem[b]).start(add=True)

        def wait_scatter(b):
            pltpu.make_async_copy(rows[b], accum.at[didx.at[0]], ssem[b]).wait()

        load_idx_span(0)
        _strided_row_chunks(
            n_nodes, sid, NS,
            lambda r: pltpu.sync_copy(zeros_v, accum.at[pl.ds(r, ZROWS)]))
        plsc.subcore_barrier()

        @pl.loop(0, n_phases)
        def _(ph):
            @pl.when(ph > 0)
            def _():
                load_idx_span(ph)

            start_gather(0, 0)

            @pl.loop(0, span // 2)
            def _(g):
                j0 = 2 * g
                wait_gather(0)
                start_scatter(j0, 0)

                @pl.when(g > 0)
                def _():
                    wait_scatter(1)

                start_gather(j0 + 1, 1)
                wait_gather(1)
                start_scatter(j0 + 1, 1)
                wait_scatter(0)

                @pl.when(g < span // 2 - 1)
                def _():
                    start_gather(j0 + 2, 0)

            wait_scatter(1)

        plsc.subcore_barrier()
        _strided_row_chunks(
            n_nodes, sid, NS,
            lambda r: pltpu.sync_copy(
                accum.at[pl.ds(r, ZROWS)],
                out_hbm.at[pl.ds(cid * n_nodes + r, ZROWS)]))

    return agg_kernel


def _norm_from_deg(deg_ref):
    return lax.rsqrt(jnp.maximum(deg_ref[:, 0], 1.0))


def _tc_pre_body(x_ref, degs_ref, w_ref, o_ref):
    norm = _norm_from_deg(degs_ref)
    o_ref[...] = jnp.dot(x_ref[...] * norm[:, None], w_ref[...],
                         preferred_element_type=jnp.float32)


def _tc_mid_body(p0_ref, p1_ref, degd_ref, b_ref, degs_ref, w_ref, o_ref):
    agg = p0_ref[...] + p1_ref[...]
    nd = _norm_from_deg(degd_ref)
    h = jnp.maximum(agg * nd[:, None] + b_ref[...], 0.0)
    ns = _norm_from_deg(degs_ref)
    o_ref[...] = jnp.dot(h * ns[:, None], w_ref[...],
                         preferred_element_type=jnp.float32)


def _tc_fin_body(p0_ref, p1_ref, degd_ref, b_ref, h_ref, c_ref):
    agg = p0_ref[...] + p1_ref[...]
    nd = _norm_from_deg(degd_ref)
    z = agg * nd[:, None] + b_ref[...]
    h = jax.nn.sigmoid(z)
    h_ref[...] = h
    c_ref[...] = jnp.where(h >= 0.5, 1.0, 0.0)


def kernel(in_feat, edge_index, W1, b1, W2, b2, W3, b3):
    n, f = in_feat.shape
    e = edge_index.shape[1]
    assert e % CHUNK == 0 and n % ZROWS == 0 and f % 128 == 0

    src = edge_index[0]
    dst = edge_index[1]
    idx_flat = edge_index.reshape(2 * e)

    degq = _make_degree_kernel(e, n, f)(idx_flat).reshape(NC, n, f)
    deg_src, deg_dst = degq[0], degq[1]

    span = CHUNK * 4 * NW
    e_pad = ((e + span - 1) // span) * span
    dummy_dst = n + (jnp.arange(e_pad - e, dtype=jnp.int32) % CHUNK)
    src2d = jnp.concatenate(
        [src, jnp.zeros((e_pad - e,), jnp.int32)]).reshape(e_pad // CHUNK, CHUNK)
    dst2d = jnp.concatenate(
        [dst, dummy_dst]).reshape(e_pad // CHUNK, CHUNK)

    agg = _make_aggregate_kernel(n, e_pad // CHUNK, f, n + CHUNK)

    def agg_kernel(x, s2d, d2d):
        return agg(x, s2d, d2d)

    blk = 1000
    grid = (n // blk,)

    w_spec = pl.BlockSpec((f, f), lambda i: (0, 0))
    b_spec = pl.BlockSpec((1, f), lambda i: (0, 0))
    row_spec = pl.BlockSpec((blk, f), lambda i: (i, 0))

    tc_pre = pl.pallas_call(
        _tc_pre_body,
        out_shape=jax.ShapeDtypeStruct((n, f), jnp.float32),
        grid=grid,
        in_specs=[row_spec, row_spec, w_spec],
        out_specs=row_spec,
    )

    tc_mid = pl.pallas_call(
        _tc_mid_body,
        out_shape=jax.ShapeDtypeStruct((n, f), jnp.float32),
        grid=grid,
        in_specs=[row_spec, row_spec, row_spec, b_spec, row_spec, w_spec],
        out_specs=row_spec,
    )

    tc_fin = pl.pallas_call(
        _tc_fin_body,
        out_shape=(jax.ShapeDtypeStruct((n, f), jnp.float32),
                   jax.ShapeDtypeStruct((n, f), jnp.float32)),
        grid=grid,
        in_specs=[row_spec, row_spec, row_spec, b_spec],
        out_specs=(row_spec, row_spec),
    )

    b1r = b1.reshape(1, f)
    b2r = b2.reshape(1, f)
    b3r = b3.reshape(1, f)

    feat1 = tc_pre(in_feat, deg_src, W1)
    p = agg_kernel(feat1, src2d, dst2d).reshape(NC, n, f)
    feat2 = tc_mid(p[0], p[1], deg_dst, b1r, deg_src, W2)
    p = agg_kernel(feat2, src2d, dst2d).reshape(NC, n, f)
    feat3 = tc_mid(p[0], p[1], deg_dst, b2r, deg_src, W3)
    p = agg_kernel(feat3, src2d, dst2d).reshape(NC, n, f)
    h, h_clone = tc_fin(p[0], p[1], deg_dst, b3r)
    return (h, h_clone)


# speedup vs baseline: 2.5551x; 2.5551x over previous
import functools

import jax
import jax.numpy as jnp
from jax import lax
from jax.experimental import pallas as pl
from jax.experimental.pallas import tpu as pltpu
from jax.experimental.pallas import tpu_sc as plsc

NC = 2
NS = 16
NW = NC * NS
LANES = 16
CHUNK = 128

_MESH = plsc.VectorSubcoreMesh(core_axis_name="c", subcore_axis_name="s")


def _fill_const(ref, rows, value):
    cols = ref.shape[1]

    @pl.loop(0, rows)
    def _(i):
        @pl.loop(0, cols, step=LANES)
        def _(j):
            ref[i, pl.ds(j, LANES)] = jnp.full((LANES,), value, jnp.float32)


ZROWS = 40


def _strided_row_chunks(total_rows, worker, n_workers, body):
    n_chunks = total_rows // ZROWS
    iters = (n_chunks + n_workers - 1) // n_workers

    @pl.loop(0, iters)
    def _(i):
        c = i * n_workers + worker

        @pl.when(c < n_chunks)
        def _():
            body(c * ZROWS)


def _make_degree_kernel(n_edges, n_nodes, feat):
    n_chunks = n_edges // CHUNK
    iters = (n_chunks + NS - 1) // NS
    assert n_nodes % ZROWS == 0

    @functools.partial(
        pl.kernel,
        out_type=jax.ShapeDtypeStruct((NC * n_nodes, feat), jnp.float32),
        mesh=_MESH,
        scratch_types=[
            pltpu.VMEM((CHUNK,), jnp.int32),
            pltpu.VMEM((CHUNK, feat), jnp.float32),
            pltpu.VMEM((ZROWS, feat), jnp.float32),
            pltpu.VMEM_SHARED((n_nodes, feat), jnp.float32),
        ],
    )
    def deg_kernel(idx_hbm, out_hbm, idx_v, ones_v, zeros_v, table):
        cid = lax.axis_index("c")
        sid = lax.axis_index("s")
        _fill_const(ones_v, CHUNK, 1.0)
        _fill_const(zeros_v, ZROWS, 0.0)

        _strided_row_chunks(
            n_nodes, sid, NS,
            lambda r: pltpu.sync_copy(zeros_v, table.at[pl.ds(r, ZROWS)]))
        plsc.subcore_barrier()

        @pl.loop(0, iters)
        def _(i):
            chunk = i * NS + sid

            @pl.when(chunk < n_chunks)
            def _():
                pltpu.sync_copy(
                    idx_hbm.at[pl.ds(cid * n_edges + chunk * CHUNK, CHUNK)],
                    idx_v)
                pltpu.sync_copy(ones_v, table.at[idx_v], add=True)

        plsc.subcore_barrier()
        _strided_row_chunks(
            n_nodes, sid, NS,
            lambda r: pltpu.sync_copy(
                table.at[pl.ds(r, ZROWS)],
                out_hbm.at[pl.ds(cid * n_nodes + r, ZROWS)]))

    return deg_kernel


def _make_aggregate_kernel(n_nodes, n_chunks, feat, n_acc):
    assert n_chunks % NW == 0 and n_nodes % ZROWS == 0
    cpw = n_chunks // NW
    n_phases = 2
    span = cpw // n_phases
    assert cpw % (2 * n_phases) == 0

    @functools.partial(
        pl.kernel,
        out_type=jax.ShapeDtypeStruct((NC * n_nodes, feat), jnp.float32),
        mesh=_MESH,
        scratch_types=[
            pltpu.VMEM((span, CHUNK), jnp.int32),
            pltpu.VMEM((span, CHUNK), jnp.int32),
            pltpu.VMEM((CHUNK, feat), jnp.float32),
            pltpu.VMEM((CHUNK, feat), jnp.float32),
            pltpu.VMEM((ZROWS, feat), jnp.float32),
            pltpu.VMEM_SHARED((n_acc, feat), jnp.float32),
            pltpu.SemaphoreType.DMA,
            pltpu.SemaphoreType.DMA,
            pltpu.SemaphoreType.DMA,
            pltpu.SemaphoreType.DMA,
        ],
    )
    def agg_kernel(x_hbm, src_hbm, dst_hbm, out_hbm,
                   sidx, didx, rows0, rows1, zeros_v, accum,
                   gsem0, gsem1, ssem0, ssem1):
        cid = lax.axis_index("c")
        sid = lax.axis_index("s")
        lw = cid * NS + sid
        _fill_const(zeros_v, ZROWS, 0.0)

        rows = (rows0, rows1)
        gsem = (gsem0, gsem1)
        ssem = (ssem0, ssem1)

        def load_idx_span(ph):
            base = lw * cpw + ph * span
            cp0 = pltpu.async_copy(src_hbm.at[pl.ds(base, span)], sidx, gsem0)
            cp1 = pltpu.async_copy(dst_hbm.at[pl.ds(base, span)], didx, gsem1)
            cp0.wait()
            cp1.wait()

        def start_gather(j, b):
            pltpu.make_async_copy(x_hbm.at[sidx.at[j]], rows[b], gsem[b]).start()

        def wait_gather(b):
            pltpu.make_async_copy(x_hbm.at[sidx.at[0]], rows[b], gsem[b]).wait()

        def start_scatter(j, b):
            pltpu.make_async_copy(rows[b], accum.at[didx.at[j]],
                                  ssem[b]).start(add=True)

        def wait_scatter(b):
            pltpu.make_async_copy(rows[b], accum.at[didx.at[0]], ssem[b]).wait()

        load_idx_span(0)
        _strided_row_chunks(
            n_nodes, sid, NS,
            lambda r: pltpu.sync_copy(zeros_v, accum.at[pl.ds(r, ZROWS)]))
        plsc.subcore_barrier()

        @pl.loop(0, n_phases)
        def _(ph):
            @pl.when(ph > 0)
            def _():
                load_idx_span(ph)

            start_gather(0, 0)

            @pl.loop(0, span // 2)
            def _(g):
                j0 = 2 * g
                wait_gather(0)
                start_scatter(j0, 0)

                @pl.when(g > 0)
                def _():
                    wait_scatter(1)

                start_gather(j0 + 1, 1)
                wait_gather(1)
                start_scatter(j0 + 1, 1)
                wait_scatter(0)

                @pl.when(g < span // 2 - 1)
                def _():
                    start_gather(j0 + 2, 0)

            wait_scatter(1)

        plsc.subcore_barrier()
        _strided_row_chunks(
            n_nodes, sid, NS,
            lambda r: pltpu.sync_copy(
                accum.at[pl.ds(r, ZROWS)],
                out_hbm.at[pl.ds(cid * n_nodes + r, ZROWS)]))

    return agg_kernel


def _norm_from_deg(deg_ref):
    return lax.rsqrt(jnp.maximum(deg_ref[:, 0], 1.0))


def _tc_pre_body(x_ref, degs_ref, w_ref, o_ref):
    norm = _norm_from_deg(degs_ref)
    o_ref[...] = jnp.dot(x_ref[...] * norm[:, None], w_ref[...],
                         preferred_element_type=jnp.float32)


def _tc_mid_body(p0_ref, p1_ref, degd_ref, b_ref, degs_ref, w_ref, o_ref):
    agg = p0_ref[...] + p1_ref[...]
    nd = _norm_from_deg(degd_ref)
    h = jnp.maximum(agg * nd[:, None] + b_ref[...], 0.0)
    ns = _norm_from_deg(degs_ref)
    o_ref[...] = jnp.dot(h * ns[:, None], w_ref[...],
                         preferred_element_type=jnp.float32)


def _tc_fin_body(p0_ref, p1_ref, degd_ref, b_ref, h_ref, c_ref):
    agg = p0_ref[...] + p1_ref[...]
    nd = _norm_from_deg(degd_ref)
    z = agg * nd[:, None] + b_ref[...]
    h = jax.nn.sigmoid(z)
    h_ref[...] = h
    c_ref[...] = jnp.where(h >= 0.5, 1.0, 0.0)


def kernel(in_feat, edge_index, W1, b1, W2, b2, W3, b3):
    n, f = in_feat.shape
    e = edge_index.shape[1]
    assert e % CHUNK == 0 and n % ZROWS == 0 and f % 128 == 0

    src = edge_index[0]
    dst = edge_index[1]
    idx_flat = edge_index.reshape(2 * e)

    degq = _make_degree_kernel(e, n, f)(idx_flat).reshape(NC, n, f)
    deg_src, deg_dst = degq[0], degq[1]

    span = CHUNK * 4 * NW
    e_pad = ((e + span - 1) // span) * span
    dummy_dst = n + (jnp.arange(e_pad - e, dtype=jnp.int32) % CHUNK)
    dummy_src = jnp.arange(e_pad - e, dtype=jnp.int32) % CHUNK
    src2d = jnp.concatenate(
        [src, dummy_src]).reshape(e_pad // CHUNK, CHUNK)
    dst2d = jnp.concatenate(
        [dst, dummy_dst]).reshape(e_pad // CHUNK, CHUNK)

    agg = _make_aggregate_kernel(n, e_pad // CHUNK, f, n + CHUNK)

    def agg_kernel(x, s2d, d2d):
        return agg(x, s2d, d2d)

    blk = 1000
    grid = (n // blk,)

    w_spec = pl.BlockSpec((f, f), lambda i: (0, 0))
    b_spec = pl.BlockSpec((1, f), lambda i: (0, 0))
    row_spec = pl.BlockSpec((blk, f), lambda i: (i, 0))

    tc_pre = pl.pallas_call(
        _tc_pre_body,
        out_shape=jax.ShapeDtypeStruct((n, f), jnp.float32),
        grid=grid,
        in_specs=[row_spec, row_spec, w_spec],
        out_specs=row_spec,
    )

    tc_mid = pl.pallas_call(
        _tc_mid_body,
        out_shape=jax.ShapeDtypeStruct((n, f), jnp.float32),
        grid=grid,
        in_specs=[row_spec, row_spec, row_spec, b_spec, row_spec, w_spec],
        out_specs=row_spec,
    )

    tc_fin = pl.pallas_call(
        _tc_fin_body,
        out_shape=(jax.ShapeDtypeStruct((n, f), jnp.float32),
                   jax.ShapeDtypeStruct((n, f), jnp.float32)),
        grid=grid,
        in_specs=[row_spec, row_spec, row_spec, b_spec],
        out_specs=(row_spec, row_spec),
    )

    b1r = b1.reshape(1, f)
    b2r = b2.reshape(1, f)
    b3r = b3.reshape(1, f)

    feat1 = tc_pre(in_feat, deg_src, W1)
    p = agg_kernel(feat1, src2d, dst2d).reshape(NC, n, f)
    feat2 = tc_mid(p[0], p[1], deg_dst, b1r, deg_src, W2)
    p = agg_kernel(feat2, src2d, dst2d).reshape(NC, n, f)
    feat3 = tc_mid(p[0], p[1], deg_dst, b2r, deg_src, W3)
    p = agg_kernel(feat3, src2d, dst2d).reshape(NC, n, f)
    h, h_clone = tc_fin(p[0], p[1], deg_dst, b3r)
    return (h, h_clone)


# speedup vs baseline: 3.3628x; 1.3161x over previous
import dataclasses
import functools

import jax
import jax.numpy as jnp
from jax import lax
from jax.experimental import pallas as pl
from jax.experimental.pallas import tpu as pltpu
from jax.experimental.pallas import tpu_sc as plsc

NC = 2
NS = 16
NW = NC * NS
LANES = 16
CHUNK = 128

_MESH = plsc.VectorSubcoreMesh(core_axis_name="c", subcore_axis_name="s")

_NO_LAYOUT_CP = pltpu.CompilerParams()
if "needs_layout_passes" in pltpu.CompilerParams.__dataclass_fields__:
    _NO_LAYOUT_CP = dataclasses.replace(_NO_LAYOUT_CP, needs_layout_passes=False)


def _fill_const(ref, rows, value):
    cols = ref.shape[1]

    @pl.loop(0, rows)
    def _(i):
        @pl.loop(0, cols, step=LANES)
        def _(j):
            ref[i, pl.ds(j, LANES)] = jnp.full((LANES,), value, jnp.float32)


ZROWS = 40


def _strided_row_chunks(total_rows, worker, n_workers, body):
    n_chunks = total_rows // ZROWS
    iters = (n_chunks + n_workers - 1) // n_workers

    @pl.loop(0, iters)
    def _(i):
        c = i * n_workers + worker

        @pl.when(c < n_chunks)
        def _():
            body(c * ZROWS)


def _make_degree_kernel(n_edges, n_nodes, feat):
    assert n_edges % (NS * LANES) == 0
    epw = n_edges // NS
    n_pad = ((n_nodes + NS * LANES - 1) // (NS * LANES)) * (NS * LANES)
    spw = n_pad // NS
    brows = 40
    assert spw % brows == 0 and spw % LANES == 0

    @functools.partial(
        pl.kernel,
        out_type=jax.ShapeDtypeStruct((NC * n_pad * feat,), jnp.float32),
        mesh=_MESH,
        scratch_types=[
            pltpu.VMEM((epw,), jnp.int32),
            pltpu.VMEM((n_pad,), jnp.float32),
            pltpu.VMEM((NS * spw,), jnp.float32),
            pltpu.VMEM((spw,), jnp.float32),
            pltpu.VMEM((brows * feat,), jnp.float32),
            pltpu.VMEM_SHARED((NS, n_pad), jnp.float32),
            pltpu.SemaphoreType.DMA,
        ],
        compiler_params=_NO_LAYOUT_CP,
    )
    def deg_kernel(idx_hbm, out_hbm, idx_v, hist, slices, red, brow, stage,
                   sem):
        cid = lax.axis_index("c")
        sid = lax.axis_index("s")

        idx_cp = pltpu.async_copy(
            idx_hbm.at[pl.ds(cid * n_edges + sid * epw, epw)], idx_v, sem)

        @pl.loop(0, n_pad, step=LANES)
        def _(i):
            hist[pl.ds(i, LANES)] = jnp.zeros((LANES,), jnp.float32)

        idx_cp.wait()
        ones16 = jnp.full((LANES,), 1.0, jnp.float32)

        @pl.loop(0, epw, step=LANES)
        def _(i):
            plsc.addupdate_scatter(hist, [idx_v[pl.ds(i, LANES)]], ones16)

        pltpu.sync_copy(hist, stage.at[sid])
        plsc.subcore_barrier()

        base = sid * spw
        for j in range(NS):
            pltpu.sync_copy(stage.at[j, pl.ds(base, spw)],
                            slices.at[pl.ds(j * spw, spw)])

        @pl.loop(0, spw, step=LANES)
        def _(i):
            acc = slices[pl.ds(i, LANES)]
            for j in range(1, NS):
                acc = acc + slices[pl.ds(j * spw + i, LANES)]
            red[pl.ds(i, LANES)] = acc

        @pl.loop(0, spw // brows)
        def _(bi):
            @pl.loop(0, brows)
            def _(k):
                kvec = jnp.full((LANES,), 0, jnp.int32) + (bi * brows + k)
                val = plsc.load_gather(red, [kvec])

                @pl.loop(0, feat, step=LANES)
                def _(c):
                    brow[pl.ds(k * feat + c, LANES)] = val

            pltpu.sync_copy(
                brow,
                out_hbm.at[pl.ds((cid * n_pad + base + bi * brows) * feat,
                                 brows * feat)])

    return deg_kernel


def _make_aggregate_kernel(n_nodes, n_chunks, feat, n_acc):
    assert n_chunks % NW == 0 and n_nodes % ZROWS == 0
    cpw = n_chunks // NW
    n_phases = 2
    span = cpw // n_phases
    assert cpw % (2 * n_phases) == 0

    @functools.partial(
        pl.kernel,
        out_type=jax.ShapeDtypeStruct((NC * n_nodes, feat), jnp.float32),
        mesh=_MESH,
        scratch_types=[
            pltpu.VMEM((span, CHUNK), jnp.int32),
            pltpu.VMEM((span, CHUNK), jnp.int32),
            pltpu.VMEM((CHUNK, feat), jnp.float32),
            pltpu.VMEM((CHUNK, feat), jnp.float32),
            pltpu.VMEM((ZROWS, feat), jnp.float32),
            pltpu.VMEM_SHARED((n_acc, feat), jnp.float32),
            pltpu.SemaphoreType.DMA,
            pltpu.SemaphoreType.DMA,
            pltpu.SemaphoreType.DMA,
            pltpu.SemaphoreType.DMA,
        ],
    )
    def agg_kernel(x_hbm, src_hbm, dst_hbm, out_hbm,
                   sidx, didx, rows0, rows1, zeros_v, accum,
                   gsem0, gsem1, ssem0, ssem1):
        cid = lax.axis_index("c")
        sid = lax.axis_index("s")
        lw = cid * NS + sid
        _fill_const(zeros_v, ZROWS, 0.0)

        rows = (rows0, rows1)
        gsem = (gsem0, gsem1)
        ssem = (ssem0, ssem1)

        def load_idx_span(ph):
            base = lw * cpw + ph * span
            cp0 = pltpu.async_copy(src_hbm.at[pl.ds(base, span)], sidx, gsem0)
            cp1 = pltpu.async_copy(dst_hbm.at[pl.ds(base, span)], didx, gsem1)
            cp0.wait()
            cp1.wait()

        def start_gather(j, b):
            pltpu.make_async_copy(x_hbm.at[sidx.at[j]], rows[b], gsem[b]).start()

        def wait_gather(b):
            pltpu.make_async_copy(x_hbm.at[sidx.at[0]], rows[b], gsem[b]).wait()

        def start_scatter(j, b):
            pltpu.make_async_copy(rows[b], accum.at[didx.at[j]],
                                  ssem[b]).start(add=True)

        def wait_scatter(b):
            pltpu.make_async_copy(rows[b], accum.at[didx.at[0]], ssem[b]).wait()

        load_idx_span(0)
        _strided_row_chunks(
            n_nodes, sid, NS,
            lambda r: pltpu.sync_copy(zeros_v, accum.at[pl.ds(r, ZROWS)]))
        plsc.subcore_barrier()

        @pl.loop(0, n_phases)
        def _(ph):
            @pl.when(ph > 0)
            def _():
                load_idx_span(ph)

            start_gather(0, 0)

            @pl.loop(0, span // 2)
            def _(g):
                j0 = 2 * g
                wait_gather(0)
                start_scatter(j0, 0)

                @pl.when(g > 0)
                def _():
                    wait_scatter(1)

                start_gather(j0 + 1, 1)
                wait_gather(1)
                start_scatter(j0 + 1, 1)
                wait_scatter(0)

                @pl.when(g < span // 2 - 1)
                def _():
                    start_gather(j0 + 2, 0)

            wait_scatter(1)

        plsc.subcore_barrier()
        _strided_row_chunks(
            n_nodes, sid, NS,
            lambda r: pltpu.sync_copy(
                accum.at[pl.ds(r, ZROWS)],
                out_hbm.at[pl.ds(cid * n_nodes + r, ZROWS)]))

    return agg_kernel


def _norm_from_deg(deg_ref):
    return lax.rsqrt(jnp.maximum(deg_ref[:, 0], 1.0))


def _tc_pre_body(x_ref, degs_ref, w_ref, o_ref):
    norm = _norm_from_deg(degs_ref)
    o_ref[...] = jnp.dot(x_ref[...] * norm[:, None], w_ref[...],
                         preferred_element_type=jnp.float32)


def _tc_mid_body(p0_ref, p1_ref, degd_ref, b_ref, degs_ref, w_ref, o_ref):
    agg = p0_ref[...] + p1_ref[...]
    nd = _norm_from_deg(degd_ref)
    h = jnp.maximum(agg * nd[:, None] + b_ref[...], 0.0)
    ns = _norm_from_deg(degs_ref)
    o_ref[...] = jnp.dot(h * ns[:, None], w_ref[...],
                         preferred_element_type=jnp.float32)


def _tc_fin_body(p0_ref, p1_ref, degd_ref, b_ref, h_ref, c_ref):
    agg = p0_ref[...] + p1_ref[...]
    nd = _norm_from_deg(degd_ref)
    z = agg * nd[:, None] + b_ref[...]
    h = jax.nn.sigmoid(z)
    h_ref[...] = h
    c_ref[...] = jnp.where(h >= 0.5, 1.0, 0.0)


def kernel(in_feat, edge_index, W1, b1, W2, b2, W3, b3):
    n, f = in_feat.shape
    e = edge_index.shape[1]
    assert e % CHUNK == 0 and n % ZROWS == 0 and f % 128 == 0

    src = edge_index[0]
    dst = edge_index[1]
    idx_flat = edge_index.reshape(2 * e)

    n_pad = ((n + NS * LANES - 1) // (NS * LANES)) * (NS * LANES)
    degq = _make_degree_kernel(e, n, f)(idx_flat).reshape(NC, n_pad, f)
    deg_src, deg_dst = degq[0], degq[1]

    span = CHUNK * 4 * NW
    e_pad = ((e + span - 1) // span) * span
    dummy_dst = n + (jnp.arange(e_pad - e, dtype=jnp.int32) % CHUNK)
    dummy_src = jnp.arange(e_pad - e, dtype=jnp.int32) % CHUNK
    src2d = jnp.concatenate(
        [src, dummy_src]).reshape(e_pad // CHUNK, CHUNK)
    dst2d = jnp.concatenate(
        [dst, dummy_dst]).reshape(e_pad // CHUNK, CHUNK)

    agg = _make_aggregate_kernel(n, e_pad // CHUNK, f, n + CHUNK)

    def agg_kernel(x, s2d, d2d):
        return agg(x, s2d, d2d)

    blk = 1000
    grid = (n // blk,)

    w_spec = pl.BlockSpec((f, f), lambda i: (0, 0))
    b_spec = pl.BlockSpec((1, f), lambda i: (0, 0))
    row_spec = pl.BlockSpec((blk, f), lambda i: (i, 0))

    tc_pre = pl.pallas_call(
        _tc_pre_body,
        out_shape=jax.ShapeDtypeStruct((n, f), jnp.float32),
        grid=grid,
        in_specs=[row_spec, row_spec, w_spec],
        out_specs=row_spec,
    )

    tc_mid = pl.pallas_call(
        _tc_mid_body,
        out_shape=jax.ShapeDtypeStruct((n, f), jnp.float32),
        grid=grid,
        in_specs=[row_spec, row_spec, row_spec, b_spec, row_spec, w_spec],
        out_specs=row_spec,
    )

    tc_fin = pl.pallas_call(
        _tc_fin_body,
        out_shape=(jax.ShapeDtypeStruct((n, f), jnp.float32),
                   jax.ShapeDtypeStruct((n, f), jnp.float32)),
        grid=grid,
        in_specs=[row_spec, row_spec, row_spec, b_spec],
        out_specs=(row_spec, row_spec),
    )

    b1r = b1.reshape(1, f)
    b2r = b2.reshape(1, f)
    b3r = b3.reshape(1, f)

    feat1 = tc_pre(in_feat, deg_src, W1)
    p = agg_kernel(feat1, src2d, dst2d).reshape(NC, n, f)
    feat2 = tc_mid(p[0], p[1], deg_dst, b1r, deg_src, W2)
    p = agg_kernel(feat2, src2d, dst2d).reshape(NC, n, f)
    feat3 = tc_mid(p[0], p[1], deg_dst, b2r, deg_src, W3)
    p = agg_kernel(feat3, src2d, dst2d).reshape(NC, n, f)
    h, h_clone = tc_fin(p[0], p[1], deg_dst, b3r)
    return (h, h_clone)
